# baseline (speedup 1.0000x reference)
import functools
import math

import jax
import jax.numpy as jnp
from jax import lax
from jax.experimental import pallas as pl
from jax.experimental.pallas import tpu as pltpu
from jax.experimental.pallas import tpu_sc as plsc


def _angular(l):
    return [(lx, ly, l - lx - ly)
            for lx in range(l, -1, -1)
            for ly in range(l - lx, -1, -1)]


_MAXL = 3
_LVECS = [v for l in range(_MAXL + 1) for v in _angular(l)]
_LIDX = {v: i for i, v in enumerate(_LVECS)}


def _mult(v):
    l = v[0] + v[1] + v[2]
    return math.factorial(l) // (
        math.factorial(v[0]) * math.factorial(v[1]) * math.factorial(v[2]))


_NL = len(_LVECS)
_NSYM = 6
_NCH = 8
_R = 8
_LANES = 128
_VL = 16
_NW = 32
_IN_ROWS = _R * _NL * _NCH
_OUT_ROWS = _R * _NSYM * _NCH

_L1 = _angular(1)
_L2 = _angular(2)
_L3 = _angular(3)


def _invariants(x):
    a1 = {v: x[_LIDX[v]] for v in _L1}
    q1 = {v: a1[v] * a1[v] for v in _L1}
    s1 = functools.reduce(lambda a, b: a + b, list(q1.values()))
    x2p = {v: (x[_LIDX[v]] if _mult(v) == 1 else x[_LIDX[v]] * float(_mult(v)))
           for v in _L2}
    s2 = functools.reduce(
        lambda a, b: a + b, [x[_LIDX[v]] * x2p[v] for v in _L2])
    g1 = functools.reduce(
        lambda a, b: a + b,
        [x[_LIDX[v]] * x[_LIDX[v]] for v in _L3 if _mult(v) == 1])
    g3 = functools.reduce(
        lambda a, b: a + b,
        [x[_LIDX[v]] * x[_LIDX[v]] for v in _L3 if _mult(v) == 3])
    q111 = x[_LIDX[(1, 1, 1)]] * x[_LIDX[(1, 1, 1)]]
    s3 = g1 + 3.0 * (g3 + 2.0 * q111)
    t4 = []
    for i in range(3):
        for j in range(i, 3):
            v3 = tuple(p + q for p, q in zip(_L1[i], _L1[j]))
            pair = q1[_L1[i]] if i == j else a1[_L1[i]] * a1[_L1[j]]
            t4.append(pair * x2p[v3])
    s4 = functools.reduce(lambda a, b: a + b, t4)
    t5 = []
    for v2 in _L2:
        inner = functools.reduce(
            lambda a, b: a + b,
            [a1[v1] * x[_LIDX[tuple(p + q for p, q in zip(v1, v2))]]
             for v1 in _L1])
        t5.append(x2p[v2] * inner)
    s5 = functools.reduce(lambda a, b: a + b, t5)
    return (x[0], s1, s2, s3, s4, s5)


def _compute_slab(in_ref, out_ref, nblocks):
    def cbody(c, carry):
        for b in range(nblocks):
            x = [in_ref[li * _NCH + c, pl.ds(_VL * b, _VL)]
                 for li in range(_NL)]
            for s, val in enumerate(_invariants(x)):
                out_ref[s * _NCH + c, pl.ds(_VL * b, _VL)] = val
        return carry
    lax.fori_loop(0, _NCH, cbody, 0)


def _sym_body(y_hbm, tail_hbm, o_hbm, a0, a1, b0, b1, tbuf,
              si0, si1, so0, so1):
    wid = lax.axis_index("s") * 2 + lax.axis_index("c")
    ntf = y_hbm.shape[1] // _LANES
    nunits = _R * ntf
    my_n = (nunits - 1 - wid) // _NW + 1

    ibufs = (a0, a1)
    obufs = (b0, b1)
    isems = (si0, si1)
    osems = (so0, so1)

    def in_slice(u):
        r_, t_ = u // ntf, u % ntf
        return y_hbm.at[pl.ds(r_ * _NL * _NCH, _NL * _NCH),
                        pl.ds(t_ * _LANES, _LANES)]

    def out_slice(u):
        r_, t_ = u // ntf, u % ntf
        return o_hbm.at[pl.ds(r_ * _NSYM * _NCH, _NSYM * _NCH),
                        pl.ds(t_ * _LANES, _LANES)]

    pltpu.async_copy(in_slice(wid), a0, si0)

    def step(i, k):
        u = wid + i * _NW

        @pl.when(i + 1 < my_n)
        def _prefetch():
            pltpu.async_copy(in_slice(u + _NW), ibufs[1 - k], isems[1 - k])

        pltpu.make_async_copy(in_slice(u), ibufs[k], isems[k]).wait()

        @pl.when(i >= 2)
        def _drain_prev_out():
            pltpu.make_async_copy(
                obufs[k], out_slice(u - 2 * _NW), osems[k]).wait()

        _compute_slab(ibufs[k], obufs[k], _LANES // _VL)
        pltpu.async_copy(obufs[k], out_slice(u), osems[k])

    def pair(j, carry):
        i0 = j * 2

        @pl.when(i0 < my_n)
        def _even():
            step(i0, 0)

        @pl.when(i0 + 1 < my_n)
        def _odd():
            step(i0 + 1, 1)

        return carry

    lax.fori_loop(0, (my_n + 1) // 2, pair, 0)

    pltpu.make_async_copy(b0, out_slice(wid), so0).wait()
    pltpu.make_async_copy(b1, out_slice(wid), so1).wait()

    if tail_hbm.shape[1] > 0:
        @pl.when(wid >= _NW - _R)
        def _tail():
            t = wid - (_NW - _R)
            pltpu.sync_copy(
                tail_hbm.at[pl.ds(t * _NL * _NCH, _NL * _NCH), :], tbuf)
            _compute_slab(tbuf, b0, 1)
            pltpu.sync_copy(
                b0, o_hbm.at[pl.ds(t * _NSYM * _NCH, _NSYM * _NCH),
                             pl.ds(ntf * _LANES, _LANES)])


def kernel(node_attr):
    n, r, nl, ch = node_attr.shape
    assert nl == _NL and ch == _NCH and r == _R
    ntf = n // _LANES
    ntail = n - ntf * _LANES
    assert ntail % _VL == 0 and ntf >= 1
    npad = (ntf + (1 if ntail else 0)) * _LANES

    y = jnp.transpose(node_attr, (1, 2, 3, 0)).reshape(_IN_ROWS, n)
    tail = lax.slice(y, (0, ntf * _LANES), (_IN_ROWS, n))

    mesh = plsc.VectorSubcoreMesh(core_axis_name="c", subcore_axis_name="s")
    o2 = pl.kernel(
        _sym_body,
        out_type=jax.ShapeDtypeStruct((_OUT_ROWS, npad), jnp.float32),
        mesh=mesh,
        compiler_params=pltpu.CompilerParams(needs_layout_passes=False),
        scratch_types=[
            pltpu.VMEM((_IN_ROWS // _R, _LANES), jnp.float32),
            pltpu.VMEM((_IN_ROWS // _R, _LANES), jnp.float32),
            pltpu.VMEM((_OUT_ROWS // _R, _LANES), jnp.float32),
            pltpu.VMEM((_OUT_ROWS // _R, _LANES), jnp.float32),
            pltpu.VMEM((_IN_ROWS // _R, ntail), jnp.float32),
            pltpu.SemaphoreType.DMA,
            pltpu.SemaphoreType.DMA,
            pltpu.SemaphoreType.DMA,
            pltpu.SemaphoreType.DMA,
        ],
    )(y, tail)
    out = o2[:, :n].reshape(_R, _NSYM, _NCH, n).transpose(3, 0, 1, 2)
    return out


# speedup vs baseline: 1.4057x; 1.4057x over previous
import functools
import math

import jax
import jax.numpy as jnp
from jax import lax
from jax.experimental import pallas as pl
from jax.experimental.pallas import tpu as pltpu
from jax.experimental.pallas import tpu_sc as plsc


def _angular(l):
    return [(lx, ly, l - lx - ly)
            for lx in range(l, -1, -1)
            for ly in range(l - lx, -1, -1)]


_MAXL = 3
_LVECS = [v for l in range(_MAXL + 1) for v in _angular(l)]
_LIDX = {v: i for i, v in enumerate(_LVECS)}


def _mult(v):
    l = v[0] + v[1] + v[2]
    return math.factorial(l) // (
        math.factorial(v[0]) * math.factorial(v[1]) * math.factorial(v[2]))


_NL = len(_LVECS)
_NSYM = 6
_NCH = 8
_R = 8
_LANES = 256
_VL = 16
_NW = 32
_IN_ROWS = _R * _NL * _NCH
_OUT_ROWS = _R * _NSYM * _NCH

_L1 = _angular(1)
_L2 = _angular(2)
_L3 = _angular(3)


def _invariants(x):
    x2p = {v: (x[_LIDX[v]] if _mult(v) == 1 else x[_LIDX[v]] * float(_mult(v)))
           for v in _L2}
    x3p = {v: (x[_LIDX[v]] if _mult(v) == 1 else x[_LIDX[v]] * float(_mult(v)))
           for v in _L3}
    s1 = functools.reduce(
        lambda a, b: a + b, [x[_LIDX[v]] * x[_LIDX[v]] for v in _L1])
    s2 = functools.reduce(
        lambda a, b: a + b, [x[_LIDX[v]] * x2p[v] for v in _L2])
    s3 = functools.reduce(
        lambda a, b: a + b, [x[_LIDX[v]] * x3p[v] for v in _L3])
    t4 = []
    for i in range(3):
        for j in range(i, 3):
            v3 = tuple(p + q for p, q in zip(_L1[i], _L1[j]))
            t4.append(x[_LIDX[_L1[i]]] * x[_LIDX[_L1[j]]] * x2p[v3])
    s4 = functools.reduce(lambda a, b: a + b, t4)
    t5 = []
    for v1 in _L1:
        for v2 in _L2:
            v3 = tuple(p + q for p, q in zip(v1, v2))
            t5.append(x[_LIDX[v1]] * x2p[v2] * x[_LIDX[v3]])
    s5 = functools.reduce(lambda a, b: a + b, t5)
    return (x[0], s1, s2, s3, s4, s5)


def _compute_slab(in_ref, out_ref, nblocks):
    def cbody(c, carry):
        for b in range(nblocks):
            x = [in_ref[li * _NCH + c, pl.ds(_VL * b, _VL)]
                 for li in range(_NL)]
            for s, val in enumerate(_invariants(x)):
                out_ref[s * _NCH + c, pl.ds(_VL * b, _VL)] = val
        return carry
    lax.fori_loop(0, _NCH, cbody, 0)


def _sym_body(y_hbm, tail_hbm, o_hbm, a0, a1, b0, b1, tbuf,
              si0, si1, so0, so1):
    wid = lax.axis_index("s") * 2 + lax.axis_index("c")
    ntf = y_hbm.shape[1] // _LANES
    nunits = _R * ntf
    my_n = (nunits - 1 - wid) // _NW + 1

    ibufs = (a0, a1)
    obufs = (b0, b1)
    isems = (si0, si1)
    osems = (so0, so1)

    def in_slice(u):
        r_, t_ = u // ntf, u % ntf
        return y_hbm.at[pl.ds(r_ * _NL * _NCH, _NL * _NCH),
                        pl.ds(t_ * _LANES, _LANES)]

    def out_slice(u):
        r_, t_ = u // ntf, u % ntf
        return o_hbm.at[pl.ds(r_ * _NSYM * _NCH, _NSYM * _NCH),
                        pl.ds(t_ * _LANES, _LANES)]

    pltpu.async_copy(in_slice(wid), a0, si0)

    def step(i, k):
        u = wid + i * _NW

        @pl.when(i + 1 < my_n)
        def _prefetch():
            pltpu.async_copy(in_slice(u + _NW), ibufs[1 - k], isems[1 - k])

        pltpu.make_async_copy(in_slice(u), ibufs[k], isems[k]).wait()

        @pl.when(i >= 2)
        def _drain_prev_out():
            pltpu.make_async_copy(
                obufs[k], out_slice(u - 2 * _NW), osems[k]).wait()

        _compute_slab(ibufs[k], obufs[k], _LANES // _VL)
        pltpu.async_copy(obufs[k], out_slice(u), osems[k])

    def pair(j, carry):
        i0 = j * 2

        @pl.when(i0 < my_n)
        def _even():
            step(i0, 0)

        @pl.when(i0 + 1 < my_n)
        def _odd():
            step(i0 + 1, 1)

        return carry

    lax.fori_loop(0, (my_n + 1) // 2, pair, 0)

    pltpu.make_async_copy(b0, out_slice(wid), so0).wait()
    pltpu.make_async_copy(b1, out_slice(wid), so1).wait()

    if tail_hbm.shape[1] > 0:
        @pl.when(wid >= _NW - _R)
        def _tail():
            t = wid - (_NW - _R)
            pltpu.sync_copy(
                tail_hbm.at[pl.ds(t * _NL * _NCH, _NL * _NCH), :], tbuf)
            _compute_slab(tbuf, b0, 1)
            pltpu.sync_copy(
                b0, o_hbm.at[pl.ds(t * _NSYM * _NCH, _NSYM * _NCH),
                             pl.ds(ntf * _LANES, _LANES)])


def kernel(node_attr):
    n, r, nl, ch = node_attr.shape
    assert nl == _NL and ch == _NCH and r == _R
    ntf = n // _LANES
    ntail = n - ntf * _LANES
    assert ntail % _VL == 0 and ntf >= 1
    npad = (ntf + (1 if ntail else 0)) * _LANES

    y = jnp.transpose(node_attr, (1, 2, 3, 0)).reshape(_IN_ROWS, n)
    tail = lax.slice(y, (0, ntf * _LANES), (_IN_ROWS, n))

    mesh = plsc.VectorSubcoreMesh(core_axis_name="c", subcore_axis_name="s")
    o2 = pl.kernel(
        _sym_body,
        out_type=jax.ShapeDtypeStruct((_OUT_ROWS, npad), jnp.float32),
        mesh=mesh,
        compiler_params=pltpu.CompilerParams(needs_layout_passes=False),
        scratch_types=[
            pltpu.VMEM((_IN_ROWS // _R, _LANES), jnp.float32),
            pltpu.VMEM((_IN_ROWS // _R, _LANES), jnp.float32),
            pltpu.VMEM((_OUT_ROWS // _R, _LANES), jnp.float32),
            pltpu.VMEM((_OUT_ROWS // _R, _LANES), jnp.float32),
            pltpu.VMEM((_IN_ROWS // _R, ntail), jnp.float32),
            pltpu.SemaphoreType.DMA,
            pltpu.SemaphoreType.DMA,
            pltpu.SemaphoreType.DMA,
            pltpu.SemaphoreType.DMA,
        ],
    )(y, tail)
    out = o2[:, :n].reshape(_R, _NSYM, _NCH, n).transpose(3, 0, 1, 2)
    return out


# speedup vs baseline: 1.7891x; 1.2728x over previous
import functools
import math

import jax
import jax.numpy as jnp
from jax import lax
from jax.experimental import pallas as pl
from jax.experimental.pallas import tpu as pltpu
from jax.experimental.pallas import tpu_sc as plsc


def _angular(l):
    return [(lx, ly, l - lx - ly)
            for lx in range(l, -1, -1)
            for ly in range(l - lx, -1, -1)]


_MAXL = 3
_LVECS = [v for l in range(_MAXL + 1) for v in _angular(l)]
_LIDX = {v: i for i, v in enumerate(_LVECS)}


def _mult(v):
    l = v[0] + v[1] + v[2]
    return math.factorial(l) // (
        math.factorial(v[0]) * math.factorial(v[1]) * math.factorial(v[2]))


_NL = len(_LVECS)
_NSYM = 6
_NCH = 8
_R = 8
_LANES = 128
_VL = 16
_NW = 32
_IN_ROWS = _R * _NL * _NCH
_OUT_ROWS = _R * _NSYM * _NCH

_L1 = _angular(1)
_L2 = _angular(2)
_L3 = _angular(3)


def _invariants(x):
    x2p = {v: (x[_LIDX[v]] if _mult(v) == 1 else x[_LIDX[v]] * float(_mult(v)))
           for v in _L2}
    x3p = {v: (x[_LIDX[v]] if _mult(v) == 1 else x[_LIDX[v]] * float(_mult(v)))
           for v in _L3}
    s1 = functools.reduce(
        lambda a, b: a + b, [x[_LIDX[v]] * x[_LIDX[v]] for v in _L1])
    s2 = functools.reduce(
        lambda a, b: a + b, [x[_LIDX[v]] * x2p[v] for v in _L2])
    s3 = functools.reduce(
        lambda a, b: a + b, [x[_LIDX[v]] * x3p[v] for v in _L3])
    t4 = []
    for i in range(3):
        for j in range(i, 3):
            v3 = tuple(p + q for p, q in zip(_L1[i], _L1[j]))
            t4.append(x[_LIDX[_L1[i]]] * x[_LIDX[_L1[j]]] * x2p[v3])
    s4 = functools.reduce(lambda a, b: a + b, t4)
    t5 = []
    for v1 in _L1:
        for v2 in _L2:
            v3 = tuple(p + q for p, q in zip(v1, v2))
            t5.append(x[_LIDX[v1]] * x2p[v2] * x[_LIDX[v3]])
    s5 = functools.reduce(lambda a, b: a + b, t5)
    return (x[0], s1, s2, s3, s4, s5)


def _compute_slab(in_ref, out_ref, nblocks):
    def cbody(c, carry):
        for b in range(nblocks):
            x = [in_ref[li * _NCH + c, pl.ds(_VL * b, _VL)]
                 for li in range(_NL)]
            for s, val in enumerate(_invariants(x)):
                out_ref[s * _NCH + c, pl.ds(_VL * b, _VL)] = val
        return carry
    lax.fori_loop(0, _NCH, cbody, 0)


def _sym_body(y_hbm, tail_hbm, o_hbm, a0, a1, b0, b1, tbuf,
              si0, si1, so0, so1):
    wid = lax.axis_index("s") * 2 + lax.axis_index("c")
    ntf = y_hbm.shape[1] // _LANES
    nunits = _R * ntf
    my_n = (nunits - 1 - wid) // _NW + 1

    ibufs = (a0, a1)
    obufs = (b0, b1)
    isems = (si0, si1)
    osems = (so0, so1)

    def in_slice(u):
        r_, t_ = u // ntf, u % ntf
        return y_hbm.at[pl.ds(r_ * _NL * _NCH, _NL * _NCH),
                        pl.ds(t_ * _LANES, _LANES)]

    def out_slice(u):
        r_, t_ = u // ntf, u % ntf
        return o_hbm.at[pl.ds(r_ * _NSYM * _NCH, _NSYM * _NCH),
                        pl.ds(t_ * _LANES, _LANES)]

    pltpu.async_copy(in_slice(wid), a0, si0)

    def step(i, k):
        u = wid + i * _NW

        @pl.when(i + 1 < my_n)
        def _prefetch():
            pltpu.async_copy(in_slice(u + _NW), ibufs[1 - k], isems[1 - k])

        pltpu.make_async_copy(in_slice(u), ibufs[k], isems[k]).wait()

        @pl.when(i >= 2)
        def _drain_prev_out():
            pltpu.make_async_copy(
                obufs[k], out_slice(u - 2 * _NW), osems[k]).wait()

        _compute_slab(ibufs[k], obufs[k], _LANES // _VL)
        pltpu.async_copy(obufs[k], out_slice(u), osems[k])

    def pair(j, carry):
        i0 = j * 2

        @pl.when(i0 < my_n)
        def _even():
            step(i0, 0)

        @pl.when(i0 + 1 < my_n)
        def _odd():
            step(i0 + 1, 1)

        return carry

    lax.fori_loop(0, (my_n + 1) // 2, pair, 0)

    pltpu.make_async_copy(b0, out_slice(wid), so0).wait()
    pltpu.make_async_copy(b1, out_slice(wid), so1).wait()

    if tail_hbm.shape[1] > 0:
        @pl.when(wid >= _NW - _R)
        def _tail():
            t = wid - (_NW - _R)
            pltpu.sync_copy(
                tail_hbm.at[pl.ds(t * _NL * _NCH, _NL * _NCH), :], tbuf)
            _compute_slab(tbuf, b0, 1)
            pltpu.sync_copy(
                b0, o_hbm.at[pl.ds(t * _NSYM * _NCH, _NSYM * _NCH),
                             pl.ds(ntf * _LANES, _LANES)])


def kernel(node_attr):
    n, r, nl, ch = node_attr.shape
    assert nl == _NL and ch == _NCH and r == _R
    ntf = n // _LANES
    ntail = n - ntf * _LANES
    assert ntail % _VL == 0 and ntf >= 1
    npad = (ntf + (1 if ntail else 0)) * _LANES

    y = jnp.transpose(node_attr, (1, 2, 3, 0)).reshape(_IN_ROWS, n)
    tail = lax.slice(y, (0, ntf * _LANES), (_IN_ROWS, n))

    mesh = plsc.VectorSubcoreMesh(core_axis_name="c", subcore_axis_name="s")
    o2 = pl.kernel(
        _sym_body,
        out_type=jax.ShapeDtypeStruct((_OUT_ROWS, npad), jnp.float32),
        mesh=mesh,
        compiler_params=pltpu.CompilerParams(needs_layout_passes=False),
        scratch_types=[
            pltpu.VMEM((_IN_ROWS // _R, _LANES), jnp.float32),
            pltpu.VMEM((_IN_ROWS // _R, _LANES), jnp.float32),
            pltpu.VMEM((_OUT_ROWS // _R, _LANES), jnp.float32),
            pltpu.VMEM((_OUT_ROWS // _R, _LANES), jnp.float32),
            pltpu.VMEM((_IN_ROWS // _R, ntail), jnp.float32),
            pltpu.SemaphoreType.DMA,
            pltpu.SemaphoreType.DMA,
            pltpu.SemaphoreType.DMA,
            pltpu.SemaphoreType.DMA,
        ],
    )(y, tail)
    out = o2[:, :n].reshape(_R, _NSYM, _NCH, n).transpose(3, 0, 1, 2)
    return out


# speedup vs baseline: 1.7909x; 1.0010x over previous
import functools
import math

import jax
import jax.numpy as jnp
from jax import lax
from jax.experimental import pallas as pl
from jax.experimental.pallas import tpu as pltpu
from jax.experimental.pallas import tpu_sc as plsc


def _angular(l):
    return [(lx, ly, l - lx - ly)
            for lx in range(l, -1, -1)
            for ly in range(l - lx, -1, -1)]


_MAXL = 3
_LVECS = [v for l in range(_MAXL + 1) for v in _angular(l)]
_LIDX = {v: i for i, v in enumerate(_LVECS)}


def _mult(v):
    l = v[0] + v[1] + v[2]
    return math.factorial(l) // (
        math.factorial(v[0]) * math.factorial(v[1]) * math.factorial(v[2]))


_NL = len(_LVECS)
_NSYM = 6
_NCH = 8
_R = 8
_LANES = 128
_VL = 16
_NW = 32
_IN_ROWS = _R * _NL * _NCH
_OUT_ROWS = _R * _NSYM * _NCH

_L1 = _angular(1)
_L2 = _angular(2)
_L3 = _angular(3)


def _invariants(x):
    x2p = {v: (x[_LIDX[v]] if _mult(v) == 1 else x[_LIDX[v]] * float(_mult(v)))
           for v in _L2}
    x3p = {v: (x[_LIDX[v]] if _mult(v) == 1 else x[_LIDX[v]] * float(_mult(v)))
           for v in _L3}
    s1 = functools.reduce(
        lambda a, b: a + b, [x[_LIDX[v]] * x[_LIDX[v]] for v in _L1])
    s2 = functools.reduce(
        lambda a, b: a + b, [x[_LIDX[v]] * x2p[v] for v in _L2])
    s3 = functools.reduce(
        lambda a, b: a + b, [x[_LIDX[v]] * x3p[v] for v in _L3])
    t4 = []
    for i in range(3):
        for j in range(i, 3):
            v3 = tuple(p + q for p, q in zip(_L1[i], _L1[j]))
            t4.append(x[_LIDX[_L1[i]]] * x[_LIDX[_L1[j]]] * x2p[v3])
    s4 = functools.reduce(lambda a, b: a + b, t4)
    t5 = []
    for v1 in _L1:
        for v2 in _L2:
            v3 = tuple(p + q for p, q in zip(v1, v2))
            t5.append(x[_LIDX[v1]] * x2p[v2] * x[_LIDX[v3]])
    s5 = functools.reduce(lambda a, b: a + b, t5)
    return (x[0], s1, s2, s3, s4, s5)


def _compute_slab(in_ref, out_ref, nblocks):
    def cbody(c, carry):
        for b in range(nblocks):
            x = [in_ref[li * _NCH + c, pl.ds(_VL * b, _VL)]
                 for li in range(_NL)]
            for s, val in enumerate(_invariants(x)):
                out_ref[s * _NCH + c, pl.ds(_VL * b, _VL)] = val
        return carry
    lax.fori_loop(0, _NCH, cbody, 0)


def _sym_body(y_hbm, o_hbm, a0, a1, b0, b1, tbuf,
              si0, si1, so0, so1):
    wid = lax.axis_index("s") * 2 + lax.axis_index("c")
    ntf = y_hbm.shape[1] // _LANES
    ntail = y_hbm.shape[1] - ntf * _LANES
    nunits = _R * ntf
    my_n = (nunits - 1 - wid) // _NW + 1

    ibufs = (a0, a1)
    obufs = (b0, b1)
    isems = (si0, si1)
    osems = (so0, so1)

    def in_slice(u):
        r_, t_ = u // ntf, u % ntf
        return y_hbm.at[pl.ds(r_ * _NL * _NCH, _NL * _NCH),
                        pl.ds(t_ * _LANES, _LANES)]

    def out_slice(u):
        r_, t_ = u // ntf, u % ntf
        return o_hbm.at[pl.ds(r_ * _NSYM * _NCH, _NSYM * _NCH),
                        pl.ds(t_ * _LANES, _LANES)]

    pltpu.async_copy(in_slice(wid), a0, si0)

    def step(i, k):
        u = wid + i * _NW

        @pl.when(i + 1 < my_n)
        def _prefetch():
            pltpu.async_copy(in_slice(u + _NW), ibufs[1 - k], isems[1 - k])

        pltpu.make_async_copy(in_slice(u), ibufs[k], isems[k]).wait()

        @pl.when(i >= 2)
        def _drain_prev_out():
            pltpu.make_async_copy(
                obufs[k], out_slice(u - 2 * _NW), osems[k]).wait()

        _compute_slab(ibufs[k], obufs[k], _LANES // _VL)
        pltpu.async_copy(obufs[k], out_slice(u), osems[k])

    def pair(j, carry):
        i0 = j * 2

        @pl.when(i0 < my_n)
        def _even():
            step(i0, 0)

        @pl.when(i0 + 1 < my_n)
        def _odd():
            step(i0 + 1, 1)

        return carry

    lax.fori_loop(0, (my_n + 1) // 2, pair, 0)

    pltpu.make_async_copy(b0, out_slice(wid), so0).wait()
    pltpu.make_async_copy(b1, out_slice(wid), so1).wait()

    if ntail > 0:
        @pl.when(wid >= _NW - _R)
        def _tail():
            t = wid - (_NW - _R)
            pltpu.sync_copy(
                y_hbm.at[pl.ds(t * _NL * _NCH, _NL * _NCH),
                         pl.ds(ntf * _LANES, ntail)], tbuf)
            _compute_slab(tbuf, b0, 1)
            pltpu.sync_copy(
                b0, o_hbm.at[pl.ds(t * _NSYM * _NCH, _NSYM * _NCH),
                             pl.ds(ntf * _LANES, _LANES)])


def kernel(node_attr):
    n, r, nl, ch = node_attr.shape
    assert nl == _NL and ch == _NCH and r == _R
    ntf = n // _LANES
    ntail = n - ntf * _LANES
    assert ntail % _VL == 0 and ntf >= 1
    npad = (ntf + (1 if ntail else 0)) * _LANES

    y = jnp.transpose(node_attr, (1, 2, 3, 0)).reshape(_IN_ROWS, n)

    mesh = plsc.VectorSubcoreMesh(core_axis_name="c", subcore_axis_name="s")
    o2 = pl.kernel(
        _sym_body,
        out_type=jax.ShapeDtypeStruct((_OUT_ROWS, npad), jnp.float32),
        mesh=mesh,
        compiler_params=pltpu.CompilerParams(needs_layout_passes=False),
        scratch_types=[
            pltpu.VMEM((_IN_ROWS // _R, _LANES), jnp.float32),
            pltpu.VMEM((_IN_ROWS // _R, _LANES), jnp.float32),
            pltpu.VMEM((_OUT_ROWS // _R, _LANES), jnp.float32),
            pltpu.VMEM((_OUT_ROWS // _R, _LANES), jnp.float32),
            pltpu.VMEM((_IN_ROWS // _R, ntail), jnp.float32),
            pltpu.SemaphoreType.DMA,
            pltpu.SemaphoreType.DMA,
            pltpu.SemaphoreType.DMA,
            pltpu.SemaphoreType.DMA,
        ],
    )(y)
    out = o2[:, :n].reshape(_R, _NSYM, _NCH, n).transpose(3, 0, 1, 2)
    return out
